# baseline (speedup 1.0000x reference)
import functools

import jax
import jax.numpy as jnp
from jax import lax
from jax.experimental import pallas as pl
from jax.experimental.pallas import tpu as pltpu
from jax.experimental.pallas import tpu_sc as plsc

CLS_NUM = 100000
FEATURE_NUM = 64
BATCH = 16384

NC = 2
NS = 16
NW = NC * NS
B_PER_W = BATCH // NW
HIST_PAD = 100096


def _sc_body(labels_hbm, center_hbm, out_hbm,
             labv_my, labv_hist, zeros_v, ones_v, cntv, rows_v, rows_vp,
             hist, sem):
    cid = lax.axis_index("c")
    sid = lax.axis_index("s")
    wid = sid * NC + cid

    for k in range(4):
        pltpu.sync_copy(
            labels_hbm.at[pl.ds((wid * 4 + k) * 128, 128)], labv_my.at[k]
        )
    cps = [
        pltpu.async_copy(
            center_hbm.at[labv_my.at[k]],
            rows_v.at[pl.ds(k * 128, 128)],
            sem,
        )
        for k in range(4)
    ]

    for j in range(8):
        zeros_v[pl.ds(j * 16, 16)] = jnp.zeros((16,), jnp.float32)
        ones_v[pl.ds(j * 16, 16)] = jnp.ones((16,), jnp.float32)

    for k in range(8):
        pltpu.sync_copy(
            labels_hbm.at[pl.ds((sid * 8 + k) * 128, 128)], labv_hist.at[k]
        )

    for k in range(8):
        pltpu.sync_copy(zeros_v, hist.at[labv_hist.at[k]])
    plsc.subcore_barrier()
    for k in range(8):
        pltpu.sync_copy(ones_v, hist.at[labv_hist.at[k]], add=True)
    plsc.subcore_barrier()

    for k in range(4):
        pltpu.sync_copy(hist.at[labv_my.at[k]], cntv.at[pl.ds(k * 128, 128)])

    for cp in cps:
        cp.wait()

    def repack(i, carry):
        base = pl.multiple_of(i * 4, 4)
        for su in range(4):
            for q in range(4):
                rows_vp[base + su, pl.ds(q * 16, 16)] = (
                    rows_v[base + su, pl.ds(q * 16, 16)]
                )
        return carry

    lax.fori_loop(0, B_PER_W // 4, repack, 0)

    iota = lax.iota(jnp.int32, 16)
    c64 = jnp.full((16,), FEATURE_NUM, jnp.int32)
    for g in range(B_PER_W // 16):
        cv = cntv[pl.ds(g * 16, 16)]
        plsc.store_scatter(rows_vp, [g * 16 + iota, c64], cv)

    pltpu.sync_copy(rows_vp, out_hbm.at[pl.ds(wid * B_PER_W, B_PER_W)])


_sc_gather = functools.partial(
    pl.kernel,
    mesh=plsc.VectorSubcoreMesh(core_axis_name="c", subcore_axis_name="s"),
    compiler_params=pltpu.CompilerParams(
        use_tc_tiling_on_sc=False, needs_layout_passes=False
    ),
    out_type=[
        jax.ShapeDtypeStruct((BATCH, 128), jnp.float32),
    ],
    scratch_types=[
        pltpu.VMEM((4, 128), jnp.int32),
        pltpu.VMEM((8, 128), jnp.int32),
        pltpu.VMEM((128,), jnp.float32),
        pltpu.VMEM((128,), jnp.float32),
        pltpu.VMEM((B_PER_W,), jnp.float32),
        pltpu.VMEM((B_PER_W, FEATURE_NUM), jnp.float32),
        pltpu.VMEM((B_PER_W, 128), jnp.float32),
        pltpu.VMEM_SHARED((HIST_PAD,), jnp.float32),
        pltpu.SemaphoreType.DMA,
    ],
)(_sc_body)


TC_BLK = 2048


def _tc_body(datas_ref, slab_ref, out_ref):
    x = datas_ref[...]
    slab = slab_ref[...]
    diff = x - slab[:, :FEATURE_NUM]
    d2 = jnp.sum(diff * diff, axis=1, keepdims=True)
    cnt = slab[:, FEATURE_NUM:FEATURE_NUM + 1]
    part = jnp.sum(jnp.sqrt(d2) / cnt).reshape(1, 1)

    @pl.when(pl.program_id(0) == 0)
    def _():
        out_ref[...] = jnp.zeros_like(out_ref)

    out_ref[...] += part


_tc_tail = pl.pallas_call(
    _tc_body,
    grid=(BATCH // TC_BLK,),
    in_specs=[
        pl.BlockSpec((TC_BLK, FEATURE_NUM), lambda i: (i, 0)),
        pl.BlockSpec((TC_BLK, 128), lambda i: (i, 0)),
    ],
    out_specs=pl.BlockSpec((1, 1), lambda i: (0, 0)),
    out_shape=jax.ShapeDtypeStruct((1, 1), jnp.float32),
)


@jax.jit
def kernel(datas, labels, center):
    (slab,) = _sc_gather(labels.astype(jnp.int32), center)
    out = _tc_tail(datas, slab)
    return out[0, 0]
